# baseline (speedup 1.0000x reference)
import functools

import jax
import jax.numpy as jnp
from jax import lax
from jax.experimental import pallas as pl
from jax.experimental.pallas import tpu as pltpu
from jax.experimental.pallas import tpu_sc as plsc

B, L, V, D, H1, H2, T = 16, 2048, 20000, 768, 512, 256, 8
TP = 8
BV = 2048
NEG = -1e30


def _mlp_body(emb, w1, b1, w2, b2, w3, b3, out):
    bf = jnp.bfloat16
    x = emb[...].astype(bf)
    h = jnp.maximum(jnp.dot(x, w1[...].astype(bf),
                            preferred_element_type=jnp.float32) + b1[...], 0.0)
    h = jnp.maximum(jnp.dot(h.astype(bf), w2[...].astype(bf),
                            preferred_element_type=jnp.float32) + b2[...], 0.0)
    s = jnp.dot(h, w3[...], preferred_element_type=jnp.float32) + b3[...]
    m = jnp.max(s, axis=1, keepdims=True)
    lse = jnp.log(jnp.sum(jnp.exp(s - m), axis=1, keepdims=True)) + m
    out[...] = s - lse


def _mlp_table(emb, w1, b1, w2, b2, w3, b3):
    grid = (V + BV - 1) // BV
    return pl.pallas_call(
        _mlp_body,
        grid=(grid,),
        in_specs=[
            pl.BlockSpec((BV, D), lambda i: (i, 0)),
            pl.BlockSpec((D, H1), lambda i: (0, 0)),
            pl.BlockSpec((1, H1), lambda i: (0, 0)),
            pl.BlockSpec((H1, H2), lambda i: (0, 0)),
            pl.BlockSpec((1, H2), lambda i: (0, 0)),
            pl.BlockSpec((H2, TP), lambda i: (0, 0)),
            pl.BlockSpec((1, TP), lambda i: (0, 0)),
        ],
        out_specs=pl.BlockSpec((BV, TP), lambda i: (i, 0)),
        out_shape=jax.ShapeDtypeStruct((V, TP), jnp.float32),
    )(emb, w1, b1, w2, b2, w3, b3)


_ROWS = B * L
_CH = 128


def _gather_rows(idx3, table):
    info = plsc.get_sparse_core_info()
    nc, ns = info.num_cores, info.num_subcores
    nw = nc * ns
    per_w = _ROWS // nw
    nch = per_w // _CH
    mesh = plsc.VectorSubcoreMesh(core_axis_name="c", subcore_axis_name="s")

    @functools.partial(
        pl.kernel,
        mesh=mesh,
        out_type=jax.ShapeDtypeStruct((_ROWS, TP), jnp.float32),
        scratch_types=[
            pltpu.VMEM((per_w,), jnp.int32),
            pltpu.VMEM((per_w, TP), jnp.float32),
            pltpu.SemaphoreType.DMA,
        ],
        compiler_params=pltpu.CompilerParams(use_tc_tiling_on_sc=False),
    )
    def gath(idx_hbm, table_hbm, out_hbm, idx_v, rows_v, sem):
        wid = lax.axis_index("s") * nc + lax.axis_index("c")
        pltpu.sync_copy(idx_hbm.at[wid], idx_v)
        pltpu.async_copy(table_hbm.at[idx_v], rows_v, sem).wait()
        pltpu.sync_copy(rows_v, out_hbm.at[pl.ds(wid * per_w, per_w)])

    return gath(idx3, table)


_GCH = 128
_RN = 16
LK = 128
_NC = 32
_SC = L // _NC


def _crf_body(ltx, lt2, tgt, slx_r, bde_r, bd1_r, exp16_r, transx_r,
              startx_r, endx_r, sel_r, out):
    slx = slx_r[...]
    bde = bde_r[...]
    bd1 = bd1_r[...]
    startx = startx_r[...]
    endx = endx_r[...]

    ii = lax.broadcasted_iota(jnp.int32, (_NC, T, LK), 1)
    jj = lax.broadcasted_iota(jnp.int32, (_NC, T, LK), 2) & 7
    a0 = (ii == jj).astype(jnp.float32)
    o0 = jnp.zeros((_NC, LK), jnp.float32)
    cvec = 1 + _SC * lax.broadcasted_iota(jnp.int32, (_NC, LK), 0)

    def blk(bi, carry):
        a, o = carry
        for k in range(_RN):
            s = bi * _RN + k
            e = lt2[s]
            an = jnp.dot(a.reshape(_NC * T, LK), bde,
                         preferred_element_type=jnp.float32)
            an = an.reshape(_NC, T, LK) * jnp.exp(e)[:, None, :]
            valid = (cvec + s) < slx
            a = jnp.where(valid[:, None, :], an, a)
        asum = jnp.sum(a, axis=1)
        t = jnp.dot(asum, bd1, preferred_element_type=jnp.float32)
        return a / t[:, None, :], o + jnp.log(t)

    a, o = lax.fori_loop(0, _SC // _RN, blk, (a0, o0))

    q = jnp.exp(ltx[0:1, :] + startx)
    qo = jnp.zeros((1, LK), jnp.float32)
    for c in range(_NC):
        dg = jnp.broadcast_to(a[c][None], (B, T, LK)).reshape(LK, LK) * bd1
        q = jnp.dot(q, dg, preferred_element_type=jnp.float32)
        sq = jnp.dot(q, bd1, preferred_element_type=jnp.float32)
        q = q / sq
        qo = qo + o[c:c + 1, :] + jnp.log(sq)

    zrow = jnp.dot(q * jnp.exp(endx), bd1, preferred_element_type=jnp.float32)
    logz = jnp.log(zrow) + qo

    transx = transx_r[...]
    exp16 = exp16_r[...]
    lanemod = (lax.broadcasted_iota(jnp.int32, (_GCH, LK), 1) & 7
               ).astype(jnp.float32)
    acc = jnp.zeros((1, LK), jnp.float32)
    for c0 in range(0, L, _GCH):
        tg = jnp.dot(tgt[pl.ds(c0, _GCH)], exp16,
                     preferred_element_type=jnp.float32)
        tgn = jnp.dot(tgt[pl.ds(c0 + 1, _GCH)], exp16,
                      preferred_element_type=jnp.float32)
        ltc = ltx[pl.ds(c0, _GCH)]
        tpos = c0 + lax.broadcasted_iota(jnp.int32, (_GCH, LK), 0)
        oh = (lanemod == tg).astype(jnp.float32)
        ohn = (lanemod == tgn).astype(jnp.float32)
        part = jnp.where(tpos < slx, ltc * oh, 0.0)
        part += jnp.where(tpos == 0, startx * oh, 0.0)
        part += jnp.where(tpos == slx - 1, endx * oh, 0.0)
        pair = jnp.zeros((_GCH, LK), jnp.float32)
        for i in range(T):
            pair += jnp.where(tg == float(i), ohn * transx[i:i + 1, :], 0.0)
        part += jnp.where(tpos + 1 < slx, pair, 0.0)
        acc += jnp.sum(part, axis=0, keepdims=True)
    goldrow = jnp.dot(acc, bd1, preferred_element_type=jnp.float32)

    out[...] = jnp.dot(logz - goldrow, sel_r[...],
                       preferred_element_type=jnp.float32)


def _crf(ltx, lt2, tgt, slx, bde, bd1, exp16, transx, startx, endx, sel):
    return pl.pallas_call(
        _crf_body,
        out_shape=jax.ShapeDtypeStruct((1, B), jnp.float32),
    )(ltx, lt2, tgt, slx, bde, bd1, exp16, transx, startx, endx, sel)


def kernel(chars, seq_len, target, embed_table, W1, b1, W2, b2, W3, b3,
           trans_m, start_scores, end_scores):
    f32 = jnp.float32
    w3p = jnp.concatenate([W3, jnp.zeros((H2, TP - T), f32)], axis=1)
    b3p = jnp.concatenate([b3, jnp.zeros((TP - T,), f32)], axis=0).reshape(1, TP)
    table = _mlp_table(embed_table, W1, b1.reshape(1, H1), W2,
                       b2.reshape(1, H2), w3p, b3p)

    idx3 = chars.T.reshape(32, _ROWS // 32)
    gathered = _gather_rows(idx3, table)

    ltx = gathered.reshape(L, LK)
    lt2 = jnp.concatenate([ltx[1:], jnp.zeros((1, LK), f32)], axis=0
                          ).reshape(_NC, _SC, LK).swapaxes(0, 1)
    tgt = jnp.concatenate(
        [target.T.astype(f32), jnp.zeros((8, B), f32)], axis=0)
    slx = jnp.broadcast_to(seq_len.astype(jnp.int32)[:, None],
                           (B, T)).reshape(1, LK)
    bde = jnp.kron(jnp.eye(B, dtype=f32), jnp.exp(trans_m))
    bd1 = jnp.kron(jnp.eye(B, dtype=f32), jnp.ones((T, T), f32))
    exp16 = jnp.kron(jnp.eye(B, dtype=f32), jnp.ones((1, T), f32))
    transx = jnp.tile(trans_m, (1, B))
    startx = jnp.tile(start_scores, B).reshape(1, LK)
    endx = jnp.tile(end_scores, B).reshape(1, LK)
    sel = (jnp.arange(LK)[:, None] == (jnp.arange(B) * T)[None, :]).astype(f32)
    loss = _crf(ltx, lt2, tgt, slx, bde, bd1, exp16, transx, startx, endx, sel)
    return loss.reshape(B)


# speedup vs baseline: 1.0078x; 1.0078x over previous
import functools

import jax
import jax.numpy as jnp
from jax import lax
from jax.experimental import pallas as pl
from jax.experimental.pallas import tpu as pltpu
from jax.experimental.pallas import tpu_sc as plsc

B, L, V, D, H1, H2, T = 16, 2048, 20000, 768, 512, 256, 8
TP = 8
BV = 2048
NEG = -1e30


def _mlp_body(emb, w1, b1, w2, b2, w3, b3, out):
    bf = jnp.bfloat16
    x = emb[...].astype(bf)
    h = jnp.maximum(jnp.dot(x, w1[...].astype(bf),
                            preferred_element_type=jnp.float32) + b1[...], 0.0)
    h = jnp.maximum(jnp.dot(h.astype(bf), w2[...].astype(bf),
                            preferred_element_type=jnp.float32) + b2[...], 0.0)
    s = jnp.dot(h, w3[...], preferred_element_type=jnp.float32) + b3[...]
    m = jnp.max(s, axis=1, keepdims=True)
    lse = jnp.log(jnp.sum(jnp.exp(s - m), axis=1, keepdims=True)) + m
    out[...] = s - lse


def _mlp_table(emb, w1, b1, w2, b2, w3, b3):
    grid = (V + BV - 1) // BV
    return pl.pallas_call(
        _mlp_body,
        grid=(grid,),
        in_specs=[
            pl.BlockSpec((BV, D), lambda i: (i, 0)),
            pl.BlockSpec((D, H1), lambda i: (0, 0)),
            pl.BlockSpec((1, H1), lambda i: (0, 0)),
            pl.BlockSpec((H1, H2), lambda i: (0, 0)),
            pl.BlockSpec((1, H2), lambda i: (0, 0)),
            pl.BlockSpec((H2, TP), lambda i: (0, 0)),
            pl.BlockSpec((1, TP), lambda i: (0, 0)),
        ],
        out_specs=pl.BlockSpec((BV, TP), lambda i: (i, 0)),
        out_shape=jax.ShapeDtypeStruct((V, TP), jnp.float32),
    )(emb, w1, b1, w2, b2, w3, b3)


_ROWS = B * L
_CH = 128


def _gather_rows(idx3, table):
    info = plsc.get_sparse_core_info()
    nc, ns = info.num_cores, info.num_subcores
    nw = nc * ns
    per_w = _ROWS // nw
    nch = per_w // _CH
    mesh = plsc.VectorSubcoreMesh(core_axis_name="c", subcore_axis_name="s")

    @functools.partial(
        pl.kernel,
        mesh=mesh,
        out_type=jax.ShapeDtypeStruct((_ROWS, TP), jnp.float32),
        scratch_types=[
            pltpu.VMEM((per_w,), jnp.int32),
            pltpu.VMEM((per_w, TP), jnp.float32),
            pltpu.SemaphoreType.DMA,
        ],
        compiler_params=pltpu.CompilerParams(use_tc_tiling_on_sc=False),
    )
    def gath(idx_hbm, table_hbm, out_hbm, idx_v, rows_v, sem):
        wid = lax.axis_index("s") * nc + lax.axis_index("c")
        pltpu.sync_copy(idx_hbm.at[wid], idx_v)
        pltpu.async_copy(table_hbm.at[idx_v], rows_v, sem).wait()
        pltpu.sync_copy(rows_v, out_hbm.at[pl.ds(wid * per_w, per_w)])

    return gath(idx3, table)


_GCH = 128
_RN = 16
LK = 128
_NC = 32
_SC = L // _NC


def _goldpair_body(tgt, slx_r, exp16_r, transx_r, startx_r, endx_r, out):
    slx = slx_r[...]
    exp16 = exp16_r[...]
    transx = transx_r[...]
    startx = startx_r[...]
    endx = endx_r[...]
    lanemod = (lax.broadcasted_iota(jnp.int32, (_GCH, LK), 1) & 7
               ).astype(jnp.float32)
    acc = jnp.zeros((1, LK), jnp.float32)
    for c0 in range(0, L, _GCH):
        tg = jnp.dot(tgt[pl.ds(c0, _GCH)], exp16,
                     preferred_element_type=jnp.float32)
        tgn = jnp.dot(tgt[pl.ds(c0 + 1, _GCH)], exp16,
                      preferred_element_type=jnp.float32)
        tpos = c0 + lax.broadcasted_iota(jnp.int32, (_GCH, LK), 0)
        oh = (lanemod == tg).astype(jnp.float32)
        ohn = (lanemod == tgn).astype(jnp.float32)
        tv = jnp.zeros((_GCH, LK), jnp.float32)
        for i in range(T):
            tv += jnp.where(tg == float(i), transx[i:i + 1, :], 0.0)
        part = jnp.where(tpos == 0, startx * oh, 0.0)
        part += jnp.where(tpos == slx - 1, endx * oh, 0.0)
        part += jnp.where(tpos + 1 < slx, ohn * tv, 0.0)
        acc += jnp.sum(part, axis=0, keepdims=True)
    out[...] = acc


def _goldpair(tgt, slx, exp16, transx, startx, endx):
    return pl.pallas_call(
        _goldpair_body,
        out_shape=jax.ShapeDtypeStruct((1, LK), jnp.float32),
    )(tgt, slx, exp16, transx, startx, endx)


def _crf_body(ltx, lt2, tgt, gp, slx_r, bde_r, bd1_r, exp16_r,
              startx_r, endx_r, sel_r, out):
    slx = slx_r[...]
    bde = bde_r[...]
    bd1 = bd1_r[...]
    startx = startx_r[...]
    endx = endx_r[...]

    ii = lax.broadcasted_iota(jnp.int32, (_NC, T, LK), 1)
    jj = lax.broadcasted_iota(jnp.int32, (_NC, T, LK), 2) & 7
    a0 = (ii == jj).astype(jnp.float32)
    o0 = jnp.zeros((_NC, LK), jnp.float32)
    cvec = 1 + _SC * lax.broadcasted_iota(jnp.int32, (_NC, LK), 0)

    def blk(bi, carry):
        a, o = carry
        for k in range(_RN):
            s = bi * _RN + k
            e = lt2[s]
            an = jnp.dot(a.reshape(_NC * T, LK), bde,
                         preferred_element_type=jnp.float32)
            an = an.reshape(_NC, T, LK) * jnp.exp(e)[:, None, :]
            valid = (cvec + s) < slx
            a = jnp.where(valid[:, None, :], an, a)
        asum = jnp.sum(a, axis=1)
        t = jnp.dot(asum, bd1, preferred_element_type=jnp.float32)
        return a / t[:, None, :], o + jnp.log(t)

    a, o = lax.fori_loop(0, _SC // _RN, blk, (a0, o0))

    q = jnp.exp(ltx[0:1, :] + startx)
    qo = jnp.zeros((1, LK), jnp.float32)
    for c in range(_NC):
        dg = jnp.broadcast_to(a[c][None], (B, T, LK)).reshape(LK, LK) * bd1
        q = jnp.dot(q, dg, preferred_element_type=jnp.float32)
        sq = jnp.dot(q, bd1, preferred_element_type=jnp.float32)
        q = q / sq
        qo = qo + o[c:c + 1, :] + jnp.log(sq)

    zrow = jnp.dot(q * jnp.exp(endx), bd1, preferred_element_type=jnp.float32)
    logz = jnp.log(zrow) + qo

    exp16 = exp16_r[...]
    lanemod = (lax.broadcasted_iota(jnp.int32, (_GCH, LK), 1) & 7
               ).astype(jnp.float32)
    acc = gp[...]
    for c0 in range(0, L, _GCH):
        tg = jnp.dot(tgt[pl.ds(c0, _GCH)], exp16,
                     preferred_element_type=jnp.float32)
        ltc = ltx[pl.ds(c0, _GCH)]
        tpos = c0 + lax.broadcasted_iota(jnp.int32, (_GCH, LK), 0)
        oh = (lanemod == tg).astype(jnp.float32)
        part = jnp.where(tpos < slx, ltc * oh, 0.0)
        acc += jnp.sum(part, axis=0, keepdims=True)
    goldrow = jnp.dot(acc, bd1, preferred_element_type=jnp.float32)

    out[...] = jnp.dot(logz - goldrow, sel_r[...],
                       preferred_element_type=jnp.float32)


def _crf(ltx, lt2, tgt, gp, slx, bde, bd1, exp16, startx, endx, sel):
    return pl.pallas_call(
        _crf_body,
        out_shape=jax.ShapeDtypeStruct((1, B), jnp.float32),
    )(ltx, lt2, tgt, gp, slx, bde, bd1, exp16, startx, endx, sel)


def kernel(chars, seq_len, target, embed_table, W1, b1, W2, b2, W3, b3,
           trans_m, start_scores, end_scores):
    f32 = jnp.float32
    w3p = jnp.concatenate([W3, jnp.zeros((H2, TP - T), f32)], axis=1)
    b3p = jnp.concatenate([b3, jnp.zeros((TP - T,), f32)], axis=0).reshape(1, TP)
    table = _mlp_table(embed_table, W1, b1.reshape(1, H1), W2,
                       b2.reshape(1, H2), w3p, b3p)

    idx3 = chars.T.reshape(32, _ROWS // 32)
    gathered = _gather_rows(idx3, table)

    ltx = gathered.reshape(L, LK)
    lt2 = jnp.concatenate([ltx[1:], jnp.zeros((1, LK), f32)], axis=0
                          ).reshape(_NC, _SC, LK).swapaxes(0, 1)
    tgt = jnp.concatenate(
        [target.T.astype(f32), jnp.zeros((8, B), f32)], axis=0)
    slx = jnp.broadcast_to(seq_len.astype(jnp.int32)[:, None],
                           (B, T)).reshape(1, LK)
    bde = jnp.kron(jnp.eye(B, dtype=f32), jnp.exp(trans_m))
    bd1 = jnp.kron(jnp.eye(B, dtype=f32), jnp.ones((T, T), f32))
    exp16 = jnp.kron(jnp.eye(B, dtype=f32), jnp.ones((1, T), f32))
    transx = jnp.tile(trans_m, (1, B))
    startx = jnp.tile(start_scores, B).reshape(1, LK)
    endx = jnp.tile(end_scores, B).reshape(1, LK)
    sel = (jnp.arange(LK)[:, None] == (jnp.arange(B) * T)[None, :]).astype(f32)
    gp = _goldpair(tgt, slx, exp16, transx, startx, endx)
    loss = _crf(ltx, lt2, tgt, gp, slx, bde, bd1, exp16, startx, endx, sel)
    return loss.reshape(B)


# speedup vs baseline: 1.0911x; 1.0826x over previous
import functools

import jax
import jax.numpy as jnp
from jax import lax
from jax.experimental import pallas as pl
from jax.experimental.pallas import tpu as pltpu
from jax.experimental.pallas import tpu_sc as plsc

B, L, V, D, H1, H2, T = 16, 2048, 20000, 768, 512, 256, 8
TP = 8
BV = 2048


def _mlp_body(emb, w1, b1, w2, b2, w3, b3, out):
    bf = jnp.bfloat16
    x = emb[...].astype(bf)
    h = jnp.maximum(jnp.dot(x, w1[...].astype(bf),
                            preferred_element_type=jnp.float32) + b1[...], 0.0)
    h = jnp.maximum(jnp.dot(h.astype(bf), w2[...].astype(bf),
                            preferred_element_type=jnp.float32) + b2[...], 0.0)
    s = jnp.dot(h, w3[...], preferred_element_type=jnp.float32) + b3[...]
    m = jnp.max(s, axis=1, keepdims=True)
    lse = jnp.log(jnp.sum(jnp.exp(s - m), axis=1, keepdims=True)) + m
    out[...] = s - lse


def _mlp_table(emb, w1, b1, w2, b2, w3, b3):
    grid = (V + BV - 1) // BV
    return pl.pallas_call(
        _mlp_body,
        grid=(grid,),
        in_specs=[
            pl.BlockSpec((BV, D), lambda i: (i, 0)),
            pl.BlockSpec((D, H1), lambda i: (0, 0)),
            pl.BlockSpec((1, H1), lambda i: (0, 0)),
            pl.BlockSpec((H1, H2), lambda i: (0, 0)),
            pl.BlockSpec((1, H2), lambda i: (0, 0)),
            pl.BlockSpec((H2, TP), lambda i: (0, 0)),
            pl.BlockSpec((1, TP), lambda i: (0, 0)),
        ],
        out_specs=pl.BlockSpec((BV, TP), lambda i: (i, 0)),
        out_shape=jax.ShapeDtypeStruct((V, TP), jnp.float32),
    )(emb, w1, b1, w2, b2, w3, b3)


_ROWS = B * L


def _gather_rows(idx3, table):
    info = plsc.get_sparse_core_info()
    nc, ns = info.num_cores, info.num_subcores
    nw = nc * ns
    per_w = _ROWS // nw
    mesh = plsc.VectorSubcoreMesh(core_axis_name="c", subcore_axis_name="s")

    @functools.partial(
        pl.kernel,
        mesh=mesh,
        out_type=jax.ShapeDtypeStruct((_ROWS, TP), jnp.float32),
        scratch_types=[
            pltpu.VMEM((per_w,), jnp.int32),
            pltpu.VMEM((per_w, TP), jnp.float32),
            pltpu.SemaphoreType.DMA,
        ],
        compiler_params=pltpu.CompilerParams(use_tc_tiling_on_sc=False),
    )
    def gath(idx_hbm, table_hbm, out_hbm, idx_v, rows_v, sem):
        wid = lax.axis_index("s") * nc + lax.axis_index("c")
        pltpu.sync_copy(idx_hbm.at[wid], idx_v)
        pltpu.async_copy(table_hbm.at[idx_v], rows_v, sem).wait()
        pltpu.sync_copy(rows_v, out_hbm.at[pl.ds(wid * per_w, per_w)])

    return gath(idx3, table)


_GCH = 128
_RN = 16
LK = 128
_NC = 32
_SC = L // _NC


def _goldpair_body(tgt, slx_r, exp16_r, transx_r, startx_r, endx_r, out):
    slx = slx_r[...]
    exp16 = exp16_r[...]
    transx = transx_r[...]
    startx = startx_r[...]
    endx = endx_r[...]
    lanemod = (lax.broadcasted_iota(jnp.int32, (_GCH, LK), 1) & 7
               ).astype(jnp.float32)
    acc = jnp.zeros((1, LK), jnp.float32)
    for c0 in range(0, L, _GCH):
        tg = jnp.dot(tgt[pl.ds(c0, _GCH)], exp16,
                     preferred_element_type=jnp.float32)
        tgn = jnp.dot(tgt[pl.ds(c0 + 1, _GCH)], exp16,
                      preferred_element_type=jnp.float32)
        tpos = c0 + lax.broadcasted_iota(jnp.int32, (_GCH, LK), 0)
        oh = (lanemod == tg).astype(jnp.float32)
        ohn = (lanemod == tgn).astype(jnp.float32)
        tv = jnp.zeros((_GCH, LK), jnp.float32)
        for i in range(T):
            tv += jnp.where(tg == float(i), transx[i:i + 1, :], 0.0)
        part = jnp.where(tpos == 0, startx * oh, 0.0)
        part += jnp.where(tpos == slx - 1, endx * oh, 0.0)
        part += jnp.where(tpos + 1 < slx, ohn * tv, 0.0)
        acc += jnp.sum(part, axis=0, keepdims=True)
    out[...] = acc


def _goldpair(tgt, slx, exp16, transx, startx, endx):
    return pl.pallas_call(
        _goldpair_body,
        out_shape=jax.ShapeDtypeStruct((1, LK), jnp.float32),
    )(tgt, slx, exp16, transx, startx, endx)


def _crf_body(ltx, lt2, tgt, gp, slx_r, bde_r, bd1_r, exp16_r,
              startx_r, endx_r, sel_r, out):
    slx = slx_r[...]
    bde = bde_r[...]
    bd1 = bd1_r[...]
    startx = startx_r[...]
    endx = endx_r[...]

    ii = lax.broadcasted_iota(jnp.int32, (_NC, T, LK), 1)
    jj = lax.broadcasted_iota(jnp.int32, (_NC, T, LK), 2) & 7
    a0 = (ii == jj).astype(jnp.float32)
    o0 = jnp.zeros((_NC, LK), jnp.float32)
    cvec = 1 + _SC * lax.broadcasted_iota(jnp.int32, (_NC, LK), 0)

    def blk(bi, carry):
        a, o = carry
        for k in range(_RN):
            s = bi * _RN + k
            e = lt2[s]
            an = jnp.dot(a.reshape(_NC * T, LK), bde,
                         preferred_element_type=jnp.float32)
            an = an.reshape(_NC, T, LK) * jnp.exp(e)[:, None, :]
            valid = (cvec + s) < slx
            a = jnp.where(valid[:, None, :], an, a)
        asum = jnp.sum(a, axis=1)
        t = jnp.dot(asum, bd1, preferred_element_type=jnp.float32)
        return a / t[:, None, :], o + jnp.log(t)

    a, o = lax.fori_loop(0, _SC // _RN, blk, (a0, o0))

    q = jnp.exp(ltx[0:1, :] + startx)
    qo = jnp.zeros((1, LK), jnp.float32)
    for c in range(_NC):
        dg = jnp.broadcast_to(a[c][None], (B, T, LK)).reshape(LK, LK) * bd1
        q = jnp.dot(q, dg, preferred_element_type=jnp.float32)
        sq = jnp.dot(q, bd1, preferred_element_type=jnp.float32)
        q = q / sq
        qo = qo + o[c:c + 1, :] + jnp.log(sq)

    zrow = jnp.dot(q * jnp.exp(endx), bd1, preferred_element_type=jnp.float32)
    logz = jnp.log(zrow) + qo

    exp16 = exp16_r[...]
    lanemod = (lax.broadcasted_iota(jnp.int32, (_GCH, LK), 1) & 7
               ).astype(jnp.float32)
    acc = gp[...]
    for c0 in range(0, L, _GCH):
        tg = jnp.dot(tgt[pl.ds(c0, _GCH)], exp16,
                     preferred_element_type=jnp.float32)
        ltc = ltx[pl.ds(c0, _GCH)]
        tpos = c0 + lax.broadcasted_iota(jnp.int32, (_GCH, LK), 0)
        oh = (lanemod == tg).astype(jnp.float32)
        part = jnp.where(tpos < slx, ltc * oh, 0.0)
        acc += jnp.sum(part, axis=0, keepdims=True)
    goldrow = jnp.dot(acc, bd1, preferred_element_type=jnp.float32)

    out[...] = jnp.dot(logz - goldrow, sel_r[...],
                       preferred_element_type=jnp.float32)


def _crf(ltx, lt2, tgt, gp, slx, bde, bd1, exp16, startx, endx, sel):
    return pl.pallas_call(
        _crf_body,
        out_shape=jax.ShapeDtypeStruct((1, B), jnp.float32),
    )(ltx, lt2, tgt, gp, slx, bde, bd1, exp16, startx, endx, sel)


def kernel(chars, seq_len, target, embed_table, W1, b1, W2, b2, W3, b3,
           trans_m, start_scores, end_scores):
    f32 = jnp.float32
    w3p = jnp.concatenate([W3, jnp.zeros((H2, TP - T), f32)], axis=1)
    b3p = jnp.concatenate([b3, jnp.zeros((TP - T,), f32)], axis=0).reshape(1, TP)
    table = _mlp_table(embed_table, W1, b1.reshape(1, H1), W2,
                       b2.reshape(1, H2), w3p, b3p)

    idx3 = chars.T.reshape(32, _ROWS // 32)
    gathered = _gather_rows(idx3, table)

    ltx = gathered.reshape(L, LK)
    lt2 = jnp.concatenate([ltx[1:], jnp.zeros((1, LK), f32)], axis=0
                          ).reshape(_NC, _SC, LK).swapaxes(0, 1)
    tgt = jnp.concatenate(
        [target.T.astype(f32), jnp.zeros((8, B), f32)], axis=0)
    slx = jnp.broadcast_to(seq_len.astype(jnp.int32)[:, None],
                           (B, T)).reshape(1, LK)
    bde = jnp.kron(jnp.eye(B, dtype=f32), jnp.exp(trans_m))
    bd1 = jnp.kron(jnp.eye(B, dtype=f32), jnp.ones((T, T), f32))
    exp16 = jnp.kron(jnp.eye(B, dtype=f32), jnp.ones((1, T), f32))
    transx = jnp.tile(trans_m, (1, B))
    startx = jnp.tile(start_scores, B).reshape(1, LK)
    endx = jnp.tile(end_scores, B).reshape(1, LK)
    sel = (jnp.arange(LK)[:, None] == (jnp.arange(B) * T)[None, :]).astype(f32)
    gp = _goldpair(tgt, slx, exp16, transx, startx, endx)
    loss = _crf(ltx, lt2, tgt, gp, slx, bde, bd1, exp16, startx, endx, sel)
    return loss.reshape(B)
